# Initial kernel scaffold; baseline (speedup 1.0000x reference)
import functools

import jax
import jax.numpy as jnp
import numpy as np
from jax.experimental import pallas as pl
from jax.experimental.pallas import tpu as pltpu

N = 10000
E = 320000
D = 128
DELTA = float(np.log(2.0))
BN = 500


def _finalize_body(acc_ref, cnt_ref, deg_ref, hprev_ref, wp_ref, b_ref, out_ref):
    acc = acc_ref[...]
    s = acc[:, 0:128]
    s2 = acc[:, 128:256]
    mx = acc[:, 256:384]
    mn = acc[:, 384:512]
    cnt = cnt_ref[:, 0:1]
    has = cnt > 0.0
    cnt_c = jnp.maximum(cnt, 1.0)
    mean = s / cnt_c
    var = jnp.maximum(s2 / cnt_c - mean * mean, 0.0)
    std = jnp.sqrt(var + 1e-5)
    zero = jnp.zeros_like(mean)
    mean = jnp.where(has, mean, zero)
    std = jnp.where(has, std, jnp.full_like(std, float(np.sqrt(np.float32(1e-5)))))
    mx = jnp.where(has, mx, zero)
    mn = jnp.where(has, mn, zero)
    agg = jnp.concatenate([mean, mx, mn, std], axis=1)
    y = jnp.dot(agg, wp_ref[...], preferred_element_type=jnp.float32)
    deg = deg_ref[:, 0:1]
    ldeg = jnp.log(jnp.maximum(deg, 1.0) + 1.0)
    amp = ldeg * (1.0 / DELTA)
    att = DELTA / ldeg
    out = y[:, 0:128] + amp * y[:, 128:256] + att * y[:, 256:384]
    out_ref[...] = out + b_ref[...] + hprev_ref[...]


_finalize = pl.pallas_call(
    _finalize_body,
    grid=(N // BN,),
    in_specs=[
        pl.BlockSpec((BN, 512), lambda i: (i, 0)),
        pl.BlockSpec((BN, 16), lambda i: (i, 0)),
        pl.BlockSpec((BN, 16), lambda i: (i, 0)),
        pl.BlockSpec((BN, 128), lambda i: (i, 0)),
        pl.BlockSpec((512, 384), lambda i: (0, 0)),
        pl.BlockSpec((1, 128), lambda i: (0, 0)),
    ],
    out_specs=pl.BlockSpec((BN, 128), lambda i: (i, 0)),
    out_shape=jax.ShapeDtypeStruct((N, D), jnp.float32),
)


def kernel(x, edge_index, W0, b0, W1, b1, W2, b2):
    src = edge_index[0]
    dst = edge_index[1]
    deg = jax.ops.segment_sum(jnp.ones((E,), jnp.float32), src, num_segments=N)
    deg16 = jnp.broadcast_to(deg[:, None], (N, 16))

    h = x
    for W, b in ((W0, b0), (W1, b1), (W2, b2)):
        msg = h[src]
        ones = jnp.ones((E,), jnp.float32)
        cnt = jax.ops.segment_sum(ones, dst, num_segments=N)
        s = jax.ops.segment_sum(msg, dst, num_segments=N)
        s2 = jax.ops.segment_sum(msg * msg, dst, num_segments=N)
        mx = jax.ops.segment_max(msg, dst, num_segments=N)
        mn = jax.ops.segment_min(msg, dst, num_segments=N)
        mx = jnp.where(jnp.isfinite(mx), mx, 0.0)
        mn = jnp.where(jnp.isfinite(mn), mn, 0.0)
        acc = jnp.concatenate([s, s2, mx, mn], axis=1)
        cnt16 = jnp.broadcast_to(cnt[:, None], (N, 16))
        wp = jnp.concatenate([W[0:512], W[512:1024], W[1024:1536]], axis=1)
        h = _finalize(acc, cnt16, deg16, h, wp, b.reshape(1, 128))
    return h


# speedup vs baseline: 1.0314x; 1.0314x over previous
import functools

import jax
import jax.numpy as jnp
import numpy as np
from jax.experimental import pallas as pl
from jax.experimental.pallas import tpu as pltpu

N = 10000
E = 320000
D = 128
DELTA = float(np.log(2.0))
BN = 1000


def _finalize_body(acc_ref, cnt_ref, deg_ref, hprev_ref, wp_ref, b_ref, out_ref):
    acc = acc_ref[...]
    s = acc[:, 0:128]
    s2 = acc[:, 128:256]
    mx = acc[:, 256:384]
    mn = acc[:, 384:512]
    cnt = cnt_ref[:, 0:1]
    has = cnt > 0.0
    cnt_c = jnp.maximum(cnt, 1.0)
    mean = s / cnt_c
    var = jnp.maximum(s2 / cnt_c - mean * mean, 0.0)
    std = jnp.sqrt(var + 1e-5)
    zero = jnp.zeros_like(mean)
    mean = jnp.where(has, mean, zero)
    std = jnp.where(has, std, jnp.full_like(std, float(np.sqrt(np.float32(1e-5)))))
    mx = jnp.where(has, mx, zero)
    mn = jnp.where(has, mn, zero)
    agg = jnp.concatenate([mean, mx, mn, std], axis=1)
    y = jnp.dot(agg, wp_ref[...], preferred_element_type=jnp.float32)
    deg = deg_ref[:, 0:1]
    ldeg = jnp.log(jnp.maximum(deg, 1.0) + 1.0)
    amp = ldeg * (1.0 / DELTA)
    att = DELTA / ldeg
    out = y[:, 0:128] + amp * y[:, 128:256] + att * y[:, 256:384]
    out_ref[...] = out + b_ref[...] + hprev_ref[...]


_finalize = pl.pallas_call(
    _finalize_body,
    grid=(N // BN,),
    in_specs=[
        pl.BlockSpec((BN, 512), lambda i: (i, 0)),
        pl.BlockSpec((BN, 16), lambda i: (i, 0)),
        pl.BlockSpec((BN, 16), lambda i: (i, 0)),
        pl.BlockSpec((BN, 128), lambda i: (i, 0)),
        pl.BlockSpec((512, 384), lambda i: (0, 0)),
        pl.BlockSpec((1, 128), lambda i: (0, 0)),
    ],
    out_specs=pl.BlockSpec((BN, 128), lambda i: (i, 0)),
    out_shape=jax.ShapeDtypeStruct((N, D), jnp.float32),
)


def kernel(x, edge_index, W0, b0, W1, b1, W2, b2):
    src = edge_index[0]
    dst = edge_index[1]
    deg = jax.ops.segment_sum(jnp.ones((E,), jnp.float32), src, num_segments=N)
    deg16 = jnp.broadcast_to(deg[:, None], (N, 16))

    h = x
    for W, b in ((W0, b0), (W1, b1), (W2, b2)):
        msg = h[src]
        ones = jnp.ones((E,), jnp.float32)
        cnt = jax.ops.segment_sum(ones, dst, num_segments=N)
        s = jax.ops.segment_sum(msg, dst, num_segments=N)
        s2 = jax.ops.segment_sum(msg * msg, dst, num_segments=N)
        mx = jax.ops.segment_max(msg, dst, num_segments=N)
        mn = jax.ops.segment_min(msg, dst, num_segments=N)
        mx = jnp.where(jnp.isfinite(mx), mx, 0.0)
        mn = jnp.where(jnp.isfinite(mn), mn, 0.0)
        acc = jnp.concatenate([s, s2, mx, mn], axis=1)
        cnt16 = jnp.broadcast_to(cnt[:, None], (N, 16))
        wp = jnp.concatenate([W[0:512], W[512:1024], W[1024:1536]], axis=1)
        h = _finalize(acc, cnt16, deg16, h, wp, b.reshape(1, 128))
    return h
